# baseline (speedup 1.0000x reference)
import jax
import jax.numpy as jnp
from jax.experimental import pallas as pl
from jax.experimental.pallas import tpu as pltpu

_STRIDE = 32.0
_AW = (116.0, 156.0, 373.0)
_AH = (90.0, 198.0, 326.0)
_INV_BN = 1.0 / (1.0 + 1e-5) ** 0.5


def _yolo_body(z_ref, x_ref, wk_ref, ws_ref, wh1_ref, wh2_ref,
               g1_ref, b1_ref, g2_ref, b2_ref, g3_ref, b3_ref, bh_ref,
               out_ref, xsc_ref, ssc_ref):
    f32 = jnp.float32
    bf16 = jnp.bfloat16
    dot = lambda a, b: jax.lax.dot_general(
        a, b, (((1,), (0,)), ((), ())), preferred_element_type=f32)
    zrow = lambda n: jnp.zeros((n, 256), bf16)

    r = jax.lax.broadcasted_iota(jnp.int32, (800, 18), 0)
    c = jax.lax.broadcasted_iota(jnp.int32, (800, 18), 1)
    jv = (r % 32).astype(f32)
    iv = (r // 32).astype(f32)
    a = c // 6
    c6 = c % 6
    awv = jnp.where(a == 0, _AW[0], jnp.where(a == 1, _AW[1], _AW[2]))
    ahv = jnp.where(a == 0, _AH[0], jnp.where(a == 1, _AH[1], _AH[2]))

    for img in range(2):
        zb = jnp.transpose(z_ref[img].astype(bf16), (1, 0))
        xb = jnp.transpose(x_ref[img].astype(bf16), (1, 0))

        z8 = jnp.concatenate(
            [jnp.concatenate([zb[i * 7:(i + 1) * 7], zrow(1)], 0)
             for i in range(7)] + [zrow(16)], 0)
        x32 = jnp.concatenate(
            [jnp.concatenate([xb[i * 31:(i + 1) * 31], zrow(1)], 0)
             for i in range(31)] + [zrow(32)], 0)

        for d in range(3):
            xsc_ref[img, d] = (jnp.concatenate([x32[d:, :], zrow(d)], 0)
                               if d else x32)

        zsh = [z8[d:d + 56, :] for d in range(3)]

        acc1 = jnp.zeros((40, 256), f32)
        for di in range(3):
            for dj in range(3):
                acc1 += dot(zsh[dj][di * 8:di * 8 + 40, :], wk_ref[di, dj])
        kern = jnp.maximum(acc1 * (g1_ref[0] * _INV_BN) + b1_ref[0], 0.0)

        acc2 = jnp.zeros((928, 256), f32)
        for di in range(3):
            for dj in range(3):
                acc2 += dot(xsc_ref[img, dj, di * 32:di * 32 + 928, :],
                            ws_ref[di, dj])
        srch = jnp.maximum(acc2 * (g2_ref[0] * _INV_BN) + b2_ref[0], 0.0)

        kern_q = kern.astype(bf16).astype(f32)
        srch_q = srch.astype(bf16).astype(f32)
        for d in range(5):
            ssc_ref[img, d] = (jnp.concatenate(
                [srch_q[d:, :], jnp.zeros((d, 256), f32)], 0) if d else srch_q)
        feat = jnp.zeros((800, 256), f32)
        for i2 in range(5):
            for j2 in range(5):
                krow = kern_q[i2 * 8 + j2:i2 * 8 + j2 + 1, :]
                feat += ssc_ref[img, j2, i2 * 32:i2 * 32 + 800, :] * krow

        h = dot(feat.astype(bf16), wh1_ref[...])
        h = jnp.maximum(h * (g3_ref[0] * _INV_BN) + b3_ref[0], 0.0)
        outv = dot(h.astype(bf16), wh2_ref[...]) + bh_ref[0]

        sig = jax.nn.sigmoid(outv)
        ex = jnp.exp(outv)
        dec = jnp.where(c6 == 0, (sig + jv) * _STRIDE,
              jnp.where(c6 == 1, (sig + iv) * _STRIDE,
              jnp.where(c6 == 2, ex * awv,
              jnp.where(c6 == 3, ex * ahv, sig))))

        for anch in range(3):
            deca = dec[:, anch * 6:(anch + 1) * 6]
            for i in range(25):
                out_ref[img, anch * 625 + i * 25:anch * 625 + (i + 1) * 25, :] = (
                    deca[i * 32:i * 32 + 25, :])


def kernel(z_f, x_f, w_k, bn1_g, bn1_b, w_s, bn2_g, bn2_b,
           w_h1, bn3_g, bn3_b, w_h2, b_h2):
    B, C = z_f.shape[0], z_f.shape[1]
    f32 = jnp.float32
    bf16 = jnp.bfloat16

    z_flat = z_f.reshape(B, C, 49)
    x_flat = x_f.reshape(B, C, 961)
    wk_t = w_k.transpose(2, 3, 1, 0).astype(bf16)
    ws_t = w_s.transpose(2, 3, 1, 0).astype(bf16)
    wh1_t = w_h1[:, :, 0, 0].T.astype(bf16)
    wh2_t = w_h2[:, :, 0, 0].T.astype(bf16)

    row = lambda v: v.reshape(1, -1).astype(f32)

    bspec = lambda shp, imap: pl.BlockSpec(shp, imap)
    full0 = lambda *shp: pl.BlockSpec(shp, lambda b: (0,) * len(shp))

    out = pl.pallas_call(
        _yolo_body,
        grid=(B // 2,),
        in_specs=[
            bspec((2, C, 49), lambda b: (b, 0, 0)),
            bspec((2, C, 961), lambda b: (b, 0, 0)),
            full0(3, 3, C, C),
            full0(3, 3, C, C),
            full0(C, C),
            full0(C, 18),
            full0(1, C), full0(1, C),
            full0(1, C), full0(1, C),
            full0(1, C), full0(1, C),
            full0(1, 18),
        ],
        out_specs=pl.BlockSpec((2, 1875, 6), lambda b: (b, 0, 0)),
        out_shape=jax.ShapeDtypeStruct((B, 1875, 6), f32),
        compiler_params=pltpu.CompilerParams(
            dimension_semantics=("parallel",)),
        scratch_shapes=[
            pltpu.VMEM((2, 3, 1024, 256), jnp.bfloat16),
            pltpu.VMEM((2, 5, 928, 256), jnp.float32),
        ],
    )(z_flat, x_flat, wk_t, ws_t, wh1_t, wh2_t,
      row(bn1_g), row(bn1_b), row(bn2_g), row(bn2_b), row(bn3_g), row(bn3_b),
      row(b_h2))
    return out


# speedup vs baseline: 1.0899x; 1.0899x over previous
import jax
import jax.numpy as jnp
from jax.experimental import pallas as pl
from jax.experimental.pallas import tpu as pltpu

_STRIDE = 32.0
_AW = (116.0, 156.0, 373.0)
_AH = (90.0, 198.0, 326.0)
_INV_BN = 1.0 / (1.0 + 1e-5) ** 0.5


def _yolo_body(z_ref, x_ref, wk_ref, ws_ref, wh1_ref, wh2_ref,
               g1_ref, b1_ref, g2_ref, b2_ref, g3_ref, b3_ref, bh_ref,
               out_ref, xsc_ref, ssc_ref):
    f32 = jnp.float32
    bf16 = jnp.bfloat16
    dot = lambda a, b: jax.lax.dot_general(
        a, b, (((1,), (0,)), ((), ())), preferred_element_type=f32)
    zrow = lambda n: jnp.zeros((n, 256), bf16)

    zb = jnp.transpose(z_ref[0].astype(bf16), (1, 0))
    xb = jnp.transpose(x_ref[0].astype(bf16), (1, 0))

    z8 = jnp.concatenate(
        [jnp.concatenate([zb[i * 7:(i + 1) * 7], zrow(1)], 0) for i in range(7)]
        + [zrow(16)], 0)
    x32 = jnp.concatenate(
        [jnp.concatenate([xb[i * 31:(i + 1) * 31], zrow(1)], 0) for i in range(31)]
        + [zrow(32)], 0)

    for d in range(3):
        xsc_ref[d] = jnp.concatenate([x32[d:, :], zrow(d)], 0) if d else x32

    zsh = [z8[d:d + 56, :] for d in range(3)]

    acc1 = jnp.zeros((40, 256), f32)
    for di in range(3):
        for dj in range(3):
            acc1 += dot(zsh[dj][di * 8:di * 8 + 40, :], wk_ref[di, dj])
    kern = jnp.maximum(acc1 * (g1_ref[0] * _INV_BN) + b1_ref[0], 0.0)

    acc2 = jnp.zeros((928, 256), f32)
    for di in range(3):
        for dj in range(3):
            acc2 += dot(xsc_ref[dj, di * 32:di * 32 + 928, :],
                        ws_ref[di, dj])
    srch = jnp.maximum(acc2 * (g2_ref[0] * _INV_BN) + b2_ref[0], 0.0)

    kern_q = kern.astype(bf16).astype(f32)
    srch_q = srch.astype(bf16).astype(f32)
    for d in range(5):
        ssc_ref[d] = (jnp.concatenate([srch_q[d:, :], jnp.zeros((d, 256), f32)], 0)
                      if d else srch_q)
    feat = jnp.zeros((800, 256), f32)
    for i2 in range(5):
        for j2 in range(5):
            krow = kern_q[i2 * 8 + j2:i2 * 8 + j2 + 1, :]
            feat += ssc_ref[j2, i2 * 32:i2 * 32 + 800, :] * krow

    h = dot(feat.astype(bf16), wh1_ref[...])
    h = jnp.maximum(h * (g3_ref[0] * _INV_BN) + b3_ref[0], 0.0)
    outv = dot(h.astype(bf16), wh2_ref[...]) + bh_ref[0]

    r = jax.lax.broadcasted_iota(jnp.int32, (800, 18), 0)
    c = jax.lax.broadcasted_iota(jnp.int32, (800, 18), 1)
    jv = (r % 32).astype(f32)
    iv = (r // 32).astype(f32)
    a = c // 6
    c6 = c % 6
    awv = jnp.where(a == 0, _AW[0], jnp.where(a == 1, _AW[1], _AW[2]))
    ahv = jnp.where(a == 0, _AH[0], jnp.where(a == 1, _AH[1], _AH[2]))
    sig = jax.nn.sigmoid(outv)
    ex = jnp.exp(outv)
    dec = jnp.where(c6 == 0, (sig + jv) * _STRIDE,
          jnp.where(c6 == 1, (sig + iv) * _STRIDE,
          jnp.where(c6 == 2, ex * awv,
          jnp.where(c6 == 3, ex * ahv, sig))))

    for anch in range(3):
        deca = dec[:, anch * 6:(anch + 1) * 6]
        for i in range(25):
            out_ref[0, anch * 625 + i * 25:anch * 625 + (i + 1) * 25, :] = (
                deca[i * 32:i * 32 + 25, :])


def kernel(z_f, x_f, w_k, bn1_g, bn1_b, w_s, bn2_g, bn2_b,
           w_h1, bn3_g, bn3_b, w_h2, b_h2):
    B, C = z_f.shape[0], z_f.shape[1]
    f32 = jnp.float32

    z_flat = z_f.reshape(B, C, 49)
    x_flat = x_f.reshape(B, C, 961)
    bf16 = jnp.bfloat16
    wk_t = w_k.transpose(2, 3, 1, 0).astype(bf16)
    ws_t = w_s.transpose(2, 3, 1, 0).astype(bf16)
    wh1_t = w_h1[:, :, 0, 0].T.astype(bf16)
    wh2_t = w_h2[:, :, 0, 0].T.astype(bf16)

    row = lambda v: v.reshape(1, -1).astype(f32)

    bspec = lambda shp, imap: pl.BlockSpec(shp, imap)
    full0 = lambda *shp: pl.BlockSpec(shp, lambda b: (0,) * len(shp))

    out = pl.pallas_call(
        _yolo_body,
        grid=(B,),
        in_specs=[
            bspec((1, C, 49), lambda b: (b, 0, 0)),
            bspec((1, C, 961), lambda b: (b, 0, 0)),
            full0(3, 3, C, C),
            full0(3, 3, C, C),
            full0(C, C),
            full0(C, 18),
            full0(1, C), full0(1, C),
            full0(1, C), full0(1, C),
            full0(1, C), full0(1, C),
            full0(1, 18),
        ],
        out_specs=pl.BlockSpec((1, 1875, 6), lambda b: (b, 0, 0)),
        out_shape=jax.ShapeDtypeStruct((B, 1875, 6), f32),
        scratch_shapes=[
            pltpu.VMEM((3, 1024, 256), jnp.bfloat16),
            pltpu.VMEM((5, 928, 256), jnp.float32),
        ],
    )(z_flat, x_flat, wk_t, ws_t, wh1_t, wh2_t,
      row(bn1_g), row(bn1_b), row(bn2_g), row(bn2_b), row(bn3_g), row(bn3_b),
      row(b_h2))
    return out
